# Initial kernel scaffold; baseline (speedup 1.0000x reference)
import jax
import jax.numpy as jnp
from jax.experimental import pallas as pl


def kernel(e1_offsets, e2_offsets, e1_table, e2_table):
    raise NotImplementedError("write your pallas kernel here")


# speedup vs baseline: 5.5131x; 5.5131x over previous
import functools

import jax
import jax.numpy as jnp
from jax import lax
from jax.experimental import pallas as pl
from jax.experimental.pallas import tpu as pltpu
from jax.experimental.pallas import tpu_sc as plsc

_MAX_ABS_OFFSET = 500
_POSITION_DIM = 32
_MAX_NORM = 2.0

_CHUNK = 128


def _renorm_body(t1_ref, t2_ref, o1_ref, o2_ref):
    for t_ref, o_ref in ((t1_ref, o1_ref), (t2_ref, o2_ref)):
        t = t_ref[...]
        n = jnp.sqrt(jnp.sum(t * t, axis=1, keepdims=True))
        scale = jnp.minimum(1.0, _MAX_NORM / jnp.maximum(n, 1e-7))
        o_ref[...] = t * scale


def _renorm_tables(t1, t2):
    shape = jax.ShapeDtypeStruct(t1.shape, t1.dtype)
    return pl.pallas_call(
        _renorm_body,
        out_shape=(shape, shape),
    )(t1, t2)


def _sc_gather(t1, t2, i1, i2):
    n = i1.shape[0]
    d = t1.shape[1]
    info = plsc.get_sparse_core_info()
    nw = info.num_cores * info.num_subcores
    per_w = n // nw
    chunks = per_w // _CHUNK
    mesh = plsc.VectorSubcoreMesh(core_axis_name="c", subcore_axis_name="s")

    @functools.partial(
        pl.kernel,
        mesh=mesh,
        compiler_params=pltpu.CompilerParams(use_tc_tiling_on_sc=False),
        out_type=jax.ShapeDtypeStruct((n, 2 * d), jnp.float32),
        scratch_types=[
            pltpu.VMEM((_CHUNK,), jnp.int32),
            pltpu.VMEM((_CHUNK,), jnp.int32),
            pltpu.VMEM((_CHUNK, d), jnp.float32),
            pltpu.VMEM((_CHUNK, d), jnp.float32),
            pltpu.SemaphoreType.DMA,
            pltpu.SemaphoreType.DMA,
        ],
    )
    def run(t1_hbm, t2_hbm, i1_hbm, i2_hbm, out_hbm,
            idx1_v, idx2_v, r1_v, r2_v, sem1, sem2):
        wid = lax.axis_index("s") * info.num_cores + lax.axis_index("c")
        base0 = wid * per_w

        def step(g, carry):
            base = base0 + g * _CHUNK
            pltpu.sync_copy(i1_hbm.at[pl.ds(base, _CHUNK)], idx1_v)
            pltpu.sync_copy(i2_hbm.at[pl.ds(base, _CHUNK)], idx2_v)
            c1 = pltpu.async_copy(t1_hbm.at[idx1_v], r1_v, sem1)
            c2 = pltpu.async_copy(t2_hbm.at[idx2_v], r2_v, sem2)
            c1.wait()
            c2.wait()
            pltpu.sync_copy(r1_v, out_hbm.at[pl.ds(base, _CHUNK), pl.ds(0, d)])
            pltpu.sync_copy(r2_v, out_hbm.at[pl.ds(base, _CHUNK), pl.ds(d, d)])
            return carry

        lax.fori_loop(0, chunks, step, 0)

    return run(t1, t2, i1, i2)


def kernel(e1_offsets, e2_offsets, e1_table, e2_table):
    b, s = e1_offsets.shape
    d = e1_table.shape[1]
    t1n, t2n = _renorm_tables(
        e1_table[_MAX_ABS_OFFSET:], e2_table[_MAX_ABS_OFFSET:]
    )
    i1 = e1_offsets.reshape(-1)
    i2 = e2_offsets.reshape(-1)
    out = _sc_gather(t1n, t2n, i1, i2)
    return out.reshape(b, s, 2 * d)


# speedup vs baseline: 6.3301x; 1.1482x over previous
import functools

import jax
import jax.numpy as jnp
from jax import lax
from jax.experimental import pallas as pl
from jax.experimental.pallas import tpu as pltpu
from jax.experimental.pallas import tpu_sc as plsc

_MAX_ABS_OFFSET = 500
_POSITION_DIM = 32
_MAX_NORM = 2.0

_CHUNK = 128


def _renorm_body(t1_ref, t2_ref, o1_ref, o2_ref):
    for t_ref, o_ref in ((t1_ref, o1_ref), (t2_ref, o2_ref)):
        t = t_ref[...]
        n = jnp.sqrt(jnp.sum(t * t, axis=1, keepdims=True))
        scale = jnp.minimum(1.0, _MAX_NORM / jnp.maximum(n, 1e-7))
        o_ref[...] = t * scale


def _renorm_tables(t1, t2):
    shape = jax.ShapeDtypeStruct(t1.shape, t1.dtype)
    return pl.pallas_call(
        _renorm_body,
        out_shape=(shape, shape),
    )(t1, t2)


_ROWS_PER_STEP = 4
_GSUB = 80


def _sc_gather(t1, t2, i1, i2, out_shape):
    n = i1.shape[0]
    d = t1.shape[1]
    b, s = out_shape[0], out_shape[1]
    info = plsc.get_sparse_core_info()
    nw = info.num_cores * info.num_subcores
    rows_w = b // nw
    cpos = _ROWS_PER_STEP * s
    ng = cpos // _GSUB
    steps = rows_w // _ROWS_PER_STEP
    mesh = plsc.VectorSubcoreMesh(core_axis_name="c", subcore_axis_name="s")

    @functools.partial(
        pl.kernel,
        mesh=mesh,
        compiler_params=pltpu.CompilerParams(use_tc_tiling_on_sc=False),
        out_type=jax.ShapeDtypeStruct(out_shape, jnp.float32),
        scratch_types=[
            pltpu.VMEM((2, cpos), jnp.int32),
            pltpu.VMEM((2, cpos), jnp.int32),
            pltpu.VMEM((2, cpos, d), jnp.float32),
            pltpu.VMEM((2, cpos, d), jnp.float32),
            [pltpu.SemaphoreType.DMA] * 2,
            [pltpu.SemaphoreType.DMA] * 2,
            [pltpu.SemaphoreType.DMA] * 2,
        ],
    )
    def run(t1_hbm, t2_hbm, i1_hbm, i2_hbm, out3_hbm,
            idx1_v, idx2_v, r1_v, r2_v, sem_i, sem_g, sem_w):
        wid = lax.axis_index("s") * info.num_cores + lax.axis_index("c")
        row0 = wid * rows_w

        def issue_idx(k, slot):
            base = (row0 + k * _ROWS_PER_STEP) * s
            pltpu.async_copy(i1_hbm.at[pl.ds(base, cpos)], idx1_v.at[slot],
                             sem_i[slot])
            pltpu.async_copy(i2_hbm.at[pl.ds(base, cpos)], idx2_v.at[slot],
                             sem_i[slot])

        def wait_idx(slot):
            for iv in (idx1_v, idx2_v):
                pltpu.make_async_copy(
                    i1_hbm.at[pl.ds(0, cpos)], iv.at[slot], sem_i[slot]
                ).wait()

        def drain_writes(slot):
            for rv in (r1_v, r2_v):
                for r in range(_ROWS_PER_STEP):
                    pltpu.make_async_copy(
                        rv.at[slot, pl.ds(r * s, s), :],
                        out3_hbm.at[row0, :, pl.ds(0, d)],
                        sem_w[slot],
                    ).wait()

        issue_idx(0, 0)
        issue_idx(1, 1)

        def outer(g, carry):
            for slot in range(2):
                k = 2 * g + slot
                brow = row0 + k * _ROWS_PER_STEP

                @pl.when(g >= 1)
                def _():
                    drain_writes(slot)

                wait_idx(slot)
                handles = []
                for tv, iv, rv in ((t1_hbm, idx1_v, r1_v),
                                   (t2_hbm, idx2_v, r2_v)):
                    for j in range(ng):
                        handles.append(pltpu.async_copy(
                            tv.at[iv.at[slot, pl.ds(j * _GSUB, _GSUB)]],
                            rv.at[slot, pl.ds(j * _GSUB, _GSUB), :],
                            sem_g[slot],
                        ))
                for h in handles:
                    h.wait()
                for r in range(_ROWS_PER_STEP):
                    pltpu.async_copy(
                        r1_v.at[slot, pl.ds(r * s, s), :],
                        out3_hbm.at[brow + r, :, pl.ds(0, d)],
                        sem_w[slot],
                    )
                    pltpu.async_copy(
                        r2_v.at[slot, pl.ds(r * s, s), :],
                        out3_hbm.at[brow + r, :, pl.ds(d, d)],
                        sem_w[slot],
                    )

                @pl.when(g <= (steps // 2) - 2)
                def _():
                    issue_idx(k + 2, slot)
            return carry

        lax.fori_loop(0, steps // 2, outer, 0)
        for slot in range(2):
            drain_writes(slot)

    return run(t1, t2, i1, i2)


def kernel(e1_offsets, e2_offsets, e1_table, e2_table):
    b, s = e1_offsets.shape
    d = e1_table.shape[1]
    t1n, t2n = _renorm_tables(
        e1_table[_MAX_ABS_OFFSET:], e2_table[_MAX_ABS_OFFSET:]
    )
    i1 = e1_offsets.reshape(-1)
    i2 = e2_offsets.reshape(-1)
    return _sc_gather(t1n, t2n, i1, i2, (b, s, 2 * d))


# speedup vs baseline: 9.5074x; 1.5019x over previous
import functools

import jax
import jax.numpy as jnp
from jax import lax
from jax.experimental import pallas as pl
from jax.experimental.pallas import tpu as pltpu
from jax.experimental.pallas import tpu_sc as plsc

_MAX_ABS_OFFSET = 500
_POSITION_DIM = 32
_MAX_NORM = 2.0

_CHUNK = 128


def _renorm_body(t1_ref, t2_ref, o1_ref, o2_ref):
    for t_ref, o_ref in ((t1_ref, o1_ref), (t2_ref, o2_ref)):
        t = t_ref[...]
        n = jnp.sqrt(jnp.sum(t * t, axis=1, keepdims=True))
        scale = jnp.minimum(1.0, _MAX_NORM / jnp.maximum(n, 1e-7))
        o_ref[...] = t * scale


def _renorm_tables(t1, t2):
    shape = jax.ShapeDtypeStruct(t1.shape, t1.dtype)
    return pl.pallas_call(
        _renorm_body,
        out_shape=(shape, shape),
    )(t1, t2)


_ROWS_PER_STEP = 4
_GSUB = 80


def _sc_gather(t1, t2, i1, i2, out_shape):
    n = i1.shape[0]
    d = t1.shape[1]
    b, s = out_shape[0], out_shape[1]
    info = plsc.get_sparse_core_info()
    nw = info.num_cores * info.num_subcores
    rows_w = b // nw
    cpos = _ROWS_PER_STEP * s
    ng = cpos // _GSUB
    steps = rows_w // _ROWS_PER_STEP
    mesh = plsc.VectorSubcoreMesh(core_axis_name="c", subcore_axis_name="s")

    @functools.partial(
        pl.kernel,
        mesh=mesh,
        compiler_params=pltpu.CompilerParams(use_tc_tiling_on_sc=False),
        out_type=jax.ShapeDtypeStruct(out_shape, jnp.float32),
        scratch_types=[
            pltpu.VMEM((2, cpos), jnp.int32),
            pltpu.VMEM((2, cpos), jnp.int32),
            pltpu.VMEM((2, cpos, d), jnp.float32),
            pltpu.VMEM((2, cpos, d), jnp.float32),
            pltpu.VMEM_SHARED(t1.shape, jnp.float32),
            pltpu.VMEM_SHARED(t2.shape, jnp.float32),
            [pltpu.SemaphoreType.DMA] * 2,
            [pltpu.SemaphoreType.DMA] * 2,
            [pltpu.SemaphoreType.DMA] * 2,
        ],
    )
    def run(t1_hbm, t2_hbm, i1_hbm, i2_hbm, out3_hbm,
            idx1_v, idx2_v, r1_v, r2_v, t1_v, t2_v, sem_i, sem_g, sem_w):
        wid = lax.axis_index("s") * info.num_cores + lax.axis_index("c")
        row0 = wid * rows_w

        @pl.when(lax.axis_index("s") == 0)
        def _():
            pltpu.sync_copy(t1_hbm, t1_v)
            pltpu.sync_copy(t2_hbm, t2_v)

        plsc.subcore_barrier()

        def issue_idx(k, slot):
            base = (row0 + k * _ROWS_PER_STEP) * s
            pltpu.async_copy(i1_hbm.at[pl.ds(base, cpos)], idx1_v.at[slot],
                             sem_i[slot])
            pltpu.async_copy(i2_hbm.at[pl.ds(base, cpos)], idx2_v.at[slot],
                             sem_i[slot])

        def wait_idx(slot):
            for iv in (idx1_v, idx2_v):
                pltpu.make_async_copy(
                    i1_hbm.at[pl.ds(0, cpos)], iv.at[slot], sem_i[slot]
                ).wait()

        def drain_writes(slot):
            for rv in (r1_v, r2_v):
                for r in range(_ROWS_PER_STEP):
                    pltpu.make_async_copy(
                        rv.at[slot, pl.ds(r * s, s), :],
                        out3_hbm.at[row0, :, pl.ds(0, d)],
                        sem_w[slot],
                    ).wait()

        issue_idx(0, 0)
        issue_idx(1, 1)

        def outer(g, carry):
            for slot in range(2):
                k = 2 * g + slot
                brow = row0 + k * _ROWS_PER_STEP

                @pl.when(g >= 1)
                def _():
                    drain_writes(slot)

                wait_idx(slot)
                handles = []
                for tv, iv, rv in ((t1_v, idx1_v, r1_v),
                                   (t2_v, idx2_v, r2_v)):
                    for j in range(ng):
                        handles.append(pltpu.async_copy(
                            tv.at[iv.at[slot, pl.ds(j * _GSUB, _GSUB)]],
                            rv.at[slot, pl.ds(j * _GSUB, _GSUB), :],
                            sem_g[slot],
                        ))
                for h in handles:
                    h.wait()
                for r in range(_ROWS_PER_STEP):
                    pltpu.async_copy(
                        r1_v.at[slot, pl.ds(r * s, s), :],
                        out3_hbm.at[brow + r, :, pl.ds(0, d)],
                        sem_w[slot],
                    )
                    pltpu.async_copy(
                        r2_v.at[slot, pl.ds(r * s, s), :],
                        out3_hbm.at[brow + r, :, pl.ds(d, d)],
                        sem_w[slot],
                    )

                @pl.when(g <= (steps // 2) - 2)
                def _():
                    issue_idx(k + 2, slot)
            return carry

        lax.fori_loop(0, steps // 2, outer, 0)
        for slot in range(2):
            drain_writes(slot)

    return run(t1, t2, i1, i2)


def kernel(e1_offsets, e2_offsets, e1_table, e2_table):
    b, s = e1_offsets.shape
    d = e1_table.shape[1]
    t1n, t2n = _renorm_tables(
        e1_table[_MAX_ABS_OFFSET:], e2_table[_MAX_ABS_OFFSET:]
    )
    i1 = e1_offsets.reshape(-1)
    i2 = e2_offsets.reshape(-1)
    return _sc_gather(t1n, t2n, i1, i2, (b, s, 2 * d))


# speedup vs baseline: 10.6991x; 1.1253x over previous
import functools

import jax
import jax.numpy as jnp
from jax import lax
from jax.experimental import pallas as pl
from jax.experimental.pallas import tpu as pltpu
from jax.experimental.pallas import tpu_sc as plsc

_MAX_ABS_OFFSET = 500
_MAX_NORM = 2.0
_PAD = 512


def _prep_tables(t1, t2):
    v, d = t1.shape

    def body(t1_ref, t2_ref, o1_ref, o2_ref):
        for t_ref, o_ref in ((t1_ref, o1_ref), (t2_ref, o2_ref)):
            t = t_ref[...]
            n = jnp.sqrt(jnp.sum(t * t, axis=1, keepdims=True))
            scale = jnp.minimum(1.0, _MAX_NORM / jnp.maximum(n, 1e-7))
            tt = (t * scale).T
            ttp = jnp.concatenate(
                [tt, jnp.zeros((d, _PAD - v), jnp.float32)], axis=1
            )
            o_ref[...] = ttp.reshape(d * _PAD)

    out = jax.ShapeDtypeStruct((d * _PAD,), jnp.float32)
    return pl.pallas_call(body, out_shape=(out, out))(t1, t2)


def _sc_gather_t(tt1, tt2, i1t, i2t, d):
    s_len, b = i1t.shape
    info = plsc.get_sparse_core_info()
    nw = info.num_cores * info.num_subcores
    bw = b // nw
    nl = info.num_lanes
    nj = bw // nl
    mesh = plsc.VectorSubcoreMesh(core_axis_name="c", subcore_axis_name="s")

    @functools.partial(
        pl.kernel,
        mesh=mesh,
        compiler_params=pltpu.CompilerParams(needs_layout_passes=False),
        out_type=jax.ShapeDtypeStruct((s_len, 2 * d, b), jnp.float32),
        scratch_types=[
            pltpu.VMEM((d * _PAD,), jnp.float32),
            pltpu.VMEM((d * _PAD,), jnp.float32),
            pltpu.VMEM((s_len, bw), jnp.int32),
            pltpu.VMEM((s_len, bw), jnp.int32),
            pltpu.VMEM((2, 2 * d, bw), jnp.float32),
            [pltpu.SemaphoreType.DMA] * 2,
        ],
    )
    def run(tt1_hbm, tt2_hbm, i1_hbm, i2_hbm, out_hbm,
            tt1_v, tt2_v, idx1_v, idx2_v, obuf, sem_o):
        wid = lax.axis_index("s") * info.num_cores + lax.axis_index("c")
        b0 = wid * bw
        pltpu.sync_copy(tt1_hbm, tt1_v)
        pltpu.sync_copy(tt2_hbm, tt2_v)
        pltpu.sync_copy(i1_hbm.at[:, pl.ds(b0, bw)], idx1_v)
        pltpu.sync_copy(i2_hbm.at[:, pl.ds(b0, bw)], idx2_v)

        def drain(slot):
            pltpu.make_async_copy(
                obuf.at[slot], out_hbm.at[0, :, pl.ds(b0, bw)], sem_o[slot]
            ).wait()

        def fill_and_send(s, slot):
            for j in range(nj):
                iv1 = idx1_v[s, pl.ds(nl * j, nl)]
                iv2 = idx2_v[s, pl.ds(nl * j, nl)]
                for f in range(d):
                    obuf[slot, f, pl.ds(nl * j, nl)] = plsc.load_gather(
                        tt1_v.at[pl.ds(_PAD * f, _PAD)], [iv1]
                    )
                for f in range(d):
                    obuf[slot, d + f, pl.ds(nl * j, nl)] = plsc.load_gather(
                        tt2_v.at[pl.ds(_PAD * f, _PAD)], [iv2]
                    )
            pltpu.async_copy(
                obuf.at[slot], out_hbm.at[s, :, pl.ds(b0, bw)], sem_o[slot]
            )

        fill_and_send(0, 0)
        fill_and_send(1, 1)

        def outer(s2, carry):
            for slot in range(2):
                s = 2 * s2 + slot
                drain(slot)
                fill_and_send(s, slot)
            return carry

        lax.fori_loop(1, s_len // 2, outer, 0)
        drain(0)
        drain(1)

    return run(tt1, tt2, i1t, i2t)


def kernel(e1_offsets, e2_offsets, e1_table, e2_table):
    b, s = e1_offsets.shape
    d = e1_table.shape[1]
    tt1, tt2 = _prep_tables(
        e1_table[_MAX_ABS_OFFSET:], e2_table[_MAX_ABS_OFFSET:]
    )
    out_t = _sc_gather_t(tt1, tt2, e1_offsets.T, e2_offsets.T, d)
    return jnp.transpose(out_t, (2, 0, 1))


# speedup vs baseline: 18.0257x; 1.6848x over previous
import functools

import jax
import jax.numpy as jnp
from jax import lax
from jax.experimental import pallas as pl
from jax.experimental.pallas import tpu as pltpu
from jax.experimental.pallas import tpu_sc as plsc

_MAX_ABS_OFFSET = 500
_MAX_NORM = 2.0
_PAD = 512


def _prep_tables(t1, t2):
    v, d = t1.shape

    def body(t1_ref, t2_ref, o1_ref, o2_ref):
        for t_ref, o_ref in ((t1_ref, o1_ref), (t2_ref, o2_ref)):
            t = t_ref[...]
            n = jnp.sqrt(jnp.sum(t * t, axis=1, keepdims=True))
            scale = jnp.minimum(1.0, _MAX_NORM / jnp.maximum(n, 1e-7))
            tt = (t * scale).T
            ttp = jnp.concatenate(
                [tt, jnp.zeros((d, _PAD - v), jnp.float32)], axis=1
            )
            o_ref[...] = ttp.reshape(d * _PAD)

    out = jax.ShapeDtypeStruct((d * _PAD,), jnp.float32)
    return pl.pallas_call(body, out_shape=(out, out))(t1, t2)


def _sc_gather_t(tt1, tt2, i1t, i2t, d):
    s_len, b = i1t.shape
    info = plsc.get_sparse_core_info()
    nw = info.num_cores * info.num_subcores
    bw = b // nw
    nl = info.num_lanes
    nj = bw // nl
    mesh = plsc.VectorSubcoreMesh(core_axis_name="c", subcore_axis_name="s")

    @functools.partial(
        pl.kernel,
        mesh=mesh,
        compiler_params=pltpu.CompilerParams(needs_layout_passes=False),
        out_type=jax.ShapeDtypeStruct((s_len, 2 * d, b), jnp.float32),
        scratch_types=[
            pltpu.VMEM((d * _PAD,), jnp.float32),
            pltpu.VMEM((d * _PAD,), jnp.float32),
            pltpu.VMEM((s_len, bw), jnp.int32),
            pltpu.VMEM((s_len, bw), jnp.int32),
            pltpu.VMEM((2, 2 * d, bw), jnp.float32),
            [pltpu.SemaphoreType.DMA] * 2,
        ],
    )
    def run(tt1_hbm, tt2_hbm, i1_hbm, i2_hbm, out_hbm,
            tt1_v, tt2_v, idx1_v, idx2_v, obuf, sem_o):
        wid = lax.axis_index("s") * info.num_cores + lax.axis_index("c")
        b0 = wid * bw
        pltpu.sync_copy(tt1_hbm, tt1_v)
        pltpu.sync_copy(tt2_hbm, tt2_v)
        pltpu.sync_copy(i1_hbm.at[:, pl.ds(b0, bw)], idx1_v)
        pltpu.sync_copy(i2_hbm.at[:, pl.ds(b0, bw)], idx2_v)

        def drain(slot):
            pltpu.make_async_copy(
                obuf.at[slot], out_hbm.at[0, :, pl.ds(b0, bw)], sem_o[slot]
            ).wait()

        def fill_and_send(s, slot):
            for j in range(nj):
                iv1 = idx1_v[s, pl.ds(nl * j, nl)]
                iv2 = idx2_v[s, pl.ds(nl * j, nl)]
                for f0 in range(0, d, 4):
                    vs = []
                    for f in range(f0, f0 + 4):
                        v1 = plsc.load_gather(
                            tt1_v.at[pl.ds(_PAD * f, _PAD)], [iv1]
                        )
                        v2 = plsc.load_gather(
                            tt2_v.at[pl.ds(_PAD * f, _PAD)], [iv2]
                        )
                        vs.append((f, v1, v2))
                    for f, v1, v2 in vs:
                        obuf[slot, f, pl.ds(nl * j, nl)] = v1
                        obuf[slot, d + f, pl.ds(nl * j, nl)] = v2
            pltpu.async_copy(
                obuf.at[slot], out_hbm.at[s, :, pl.ds(b0, bw)], sem_o[slot]
            )

        fill_and_send(0, 0)
        fill_and_send(1, 1)

        def outer(s2, carry):
            for slot in range(2):
                s = 2 * s2 + slot
                drain(slot)
                fill_and_send(s, slot)
            return carry

        lax.fori_loop(1, s_len // 2, outer, 0)
        drain(0)
        drain(1)

    return run(tt1, tt2, i1t, i2t)


def kernel(e1_offsets, e2_offsets, e1_table, e2_table):
    b, s = e1_offsets.shape
    d = e1_table.shape[1]
    tt1, tt2 = _prep_tables(
        e1_table[_MAX_ABS_OFFSET:], e2_table[_MAX_ABS_OFFSET:]
    )
    out_t = _sc_gather_t(tt1, tt2, e1_offsets.T, e2_offsets.T, d)
    return jnp.transpose(out_t, (2, 0, 1))


# speedup vs baseline: 34.0703x; 1.8901x over previous
import functools

import jax
import jax.numpy as jnp
from jax import lax
from jax.experimental import pallas as pl
from jax.experimental.pallas import tpu as pltpu
from jax.experimental.pallas import tpu_sc as plsc

_MAX_ABS_OFFSET = 500
_MAX_NORM = 2.0
_PAD = 512


def _prep_tables(t1, t2):
    v, d = t1.shape

    def body(t1_ref, t2_ref, o1_ref, o2_ref):
        for t_ref, o_ref in ((t1_ref, o1_ref), (t2_ref, o2_ref)):
            t = t_ref[...]
            n = jnp.sqrt(jnp.sum(t * t, axis=1, keepdims=True))
            scale = jnp.minimum(1.0, _MAX_NORM / jnp.maximum(n, 1e-7))
            tt = (t * scale).T
            ttp = jnp.concatenate(
                [tt, jnp.zeros((d, _PAD - v), jnp.float32)], axis=1
            )
            o_ref[...] = ttp.reshape(d * _PAD)

    out = jax.ShapeDtypeStruct((d * _PAD,), jnp.float32)
    return pl.pallas_call(body, out_shape=(out, out))(t1, t2)


def _sc_gather_t(tt1, tt2, i1t, i2t, d):
    s_len, b = i1t.shape
    info = plsc.get_sparse_core_info()
    nw = info.num_cores * info.num_subcores
    bw = b // nw
    nl = info.num_lanes
    nj = bw // nl
    mesh = plsc.VectorSubcoreMesh(core_axis_name="c", subcore_axis_name="s")

    @functools.partial(
        pl.kernel,
        mesh=mesh,
        compiler_params=pltpu.CompilerParams(needs_layout_passes=False),
        out_type=jax.ShapeDtypeStruct((s_len, 2 * d, b), jnp.float32),
        scratch_types=[
            pltpu.VMEM((d * _PAD,), jnp.float32),
            pltpu.VMEM((d * _PAD,), jnp.float32),
            pltpu.VMEM((s_len, bw), jnp.int32),
            pltpu.VMEM((s_len, bw), jnp.int32),
            pltpu.VMEM((2, 2 * d, bw), jnp.float32),
            [pltpu.SemaphoreType.DMA] * 2,
        ],
    )
    def run(tt1_hbm, tt2_hbm, i1_hbm, i2_hbm, out_hbm,
            tt1_v, tt2_v, idx1_v, idx2_v, obuf, sem_o):
        wid = lax.axis_index("s") * info.num_cores + lax.axis_index("c")
        b0 = wid * bw
        pltpu.sync_copy(tt1_hbm, tt1_v)
        pltpu.sync_copy(tt2_hbm, tt2_v)
        pltpu.sync_copy(i1_hbm.at[:, pl.ds(b0, bw)], idx1_v)
        pltpu.sync_copy(i2_hbm.at[:, pl.ds(b0, bw)], idx2_v)

        def drain(slot):
            pltpu.make_async_copy(
                obuf.at[slot], out_hbm.at[0, :, pl.ds(b0, bw)], sem_o[slot]
            ).wait()

        def fill_and_send(s, slot):
            lag = 8
            for j in range(nj):
                iv1 = idx1_v[s, pl.ds(nl * j, nl)]
                iv2 = idx2_v[s, pl.ds(nl * j, nl)]
                pending = []

                def flush(j=j, slot=slot, pending=pending):
                    row, v = pending.pop(0)
                    obuf[slot, row, pl.ds(nl * j, nl)] = v

                for f in range(d):
                    for row, tv, iv in ((f, tt1_v, iv1),
                                        (d + f, tt2_v, iv2)):
                        v = plsc.load_gather(
                            tv.at[pl.ds(_PAD * f, _PAD)], [iv]
                        )
                        if len(pending) >= lag:
                            flush()
                        pending.append((row, v))
                while pending:
                    flush()
            pltpu.async_copy(
                obuf.at[slot], out_hbm.at[s, :, pl.ds(b0, bw)], sem_o[slot]
            )

        fill_and_send(0, 0)
        fill_and_send(1, 1)

        def outer(s2, carry):
            for slot in range(2):
                s = 2 * s2 + slot
                drain(slot)
                fill_and_send(s, slot)
            return carry

        lax.fori_loop(1, s_len // 2, outer, 0)
        drain(0)
        drain(1)

    return run(tt1, tt2, i1t, i2t)


def kernel(e1_offsets, e2_offsets, e1_table, e2_table):
    b, s = e1_offsets.shape
    d = e1_table.shape[1]
    tt1, tt2 = _prep_tables(
        e1_table[_MAX_ABS_OFFSET:], e2_table[_MAX_ABS_OFFSET:]
    )
    out_t = _sc_gather_t(tt1, tt2, e1_offsets.T, e2_offsets.T, d)
    return jnp.transpose(out_t, (2, 0, 1))


# speedup vs baseline: 34.3580x; 1.0084x over previous
import functools

import jax
import jax.numpy as jnp
from jax import lax
from jax.experimental import pallas as pl
from jax.experimental.pallas import tpu as pltpu
from jax.experimental.pallas import tpu_sc as plsc

_MAX_ABS_OFFSET = 500
_MAX_NORM = 2.0
_PAD = 512


def _prep_tables(t1, t2):
    v, d = t1.shape

    def body(t1_ref, t2_ref, o1_ref, o2_ref):
        for t_ref, o_ref in ((t1_ref, o1_ref), (t2_ref, o2_ref)):
            t = t_ref[...]
            n = jnp.sqrt(jnp.sum(t * t, axis=1, keepdims=True))
            scale = jnp.minimum(1.0, _MAX_NORM / jnp.maximum(n, 1e-7))
            tt = (t * scale).T
            ttp = jnp.concatenate(
                [tt, jnp.zeros((d, _PAD - v), jnp.float32)], axis=1
            )
            o_ref[...] = ttp.reshape(d * _PAD)

    out = jax.ShapeDtypeStruct((d * _PAD,), jnp.float32)
    return pl.pallas_call(body, out_shape=(out, out))(t1, t2)


def _sc_gather_t(tt1, tt2, i1t, i2t, d):
    s_len, b = i1t.shape
    info = plsc.get_sparse_core_info()
    nw = info.num_cores * info.num_subcores
    bw = b // nw
    nl = info.num_lanes
    nj = bw // nl
    mesh = plsc.VectorSubcoreMesh(core_axis_name="c", subcore_axis_name="s")

    @functools.partial(
        pl.kernel,
        mesh=mesh,
        compiler_params=pltpu.CompilerParams(needs_layout_passes=False),
        out_type=jax.ShapeDtypeStruct((s_len, 2 * d, b), jnp.float32),
        scratch_types=[
            pltpu.VMEM((d * _PAD,), jnp.float32),
            pltpu.VMEM((d * _PAD,), jnp.float32),
            pltpu.VMEM((s_len, bw), jnp.int32),
            pltpu.VMEM((s_len, bw), jnp.int32),
            pltpu.VMEM((2, 2 * d, bw), jnp.float32),
            [pltpu.SemaphoreType.DMA] * 2,
        ],
    )
    def run(tt1_hbm, tt2_hbm, i1_hbm, i2_hbm, out_hbm,
            tt1_v, tt2_v, idx1_v, idx2_v, obuf, sem_o):
        wid = lax.axis_index("s") * info.num_cores + lax.axis_index("c")
        b0 = wid * bw
        pltpu.sync_copy(tt1_hbm, tt1_v)
        pltpu.sync_copy(tt2_hbm, tt2_v)
        pltpu.sync_copy(i1_hbm.at[:, pl.ds(b0, bw)], idx1_v)
        pltpu.sync_copy(i2_hbm.at[:, pl.ds(b0, bw)], idx2_v)

        def drain(slot):
            pltpu.make_async_copy(
                obuf.at[slot], out_hbm.at[0, :, pl.ds(b0, bw)], sem_o[slot]
            ).wait()

        def fill_and_send(s, slot):
            lag = 8
            pending = []

            def flush():
                jj, row, v = pending.pop(0)
                obuf[slot, row, pl.ds(nl * jj, nl)] = v

            for j in range(nj):
                iv1 = idx1_v[s, pl.ds(nl * j, nl)]
                iv2 = idx2_v[s, pl.ds(nl * j, nl)]
                for f in range(d):
                    for row, tv, iv in ((f, tt1_v, iv1),
                                        (d + f, tt2_v, iv2)):
                        v = plsc.load_gather(
                            tv.at[pl.ds(_PAD * f, _PAD)], [iv]
                        )
                        if len(pending) >= lag:
                            flush()
                        pending.append((j, row, v))
            while pending:
                flush()
            pltpu.async_copy(
                obuf.at[slot], out_hbm.at[s, :, pl.ds(b0, bw)], sem_o[slot]
            )

        fill_and_send(0, 0)
        fill_and_send(1, 1)

        def outer(s2, carry):
            for slot in range(2):
                s = 2 * s2 + slot
                drain(slot)
                fill_and_send(s, slot)
            return carry

        lax.fori_loop(1, s_len // 2, outer, 0)
        drain(0)
        drain(1)

    return run(tt1, tt2, i1t, i2t)


def kernel(e1_offsets, e2_offsets, e1_table, e2_table):
    b, s = e1_offsets.shape
    d = e1_table.shape[1]
    tt1, tt2 = _prep_tables(
        e1_table[_MAX_ABS_OFFSET:], e2_table[_MAX_ABS_OFFSET:]
    )
    out_t = _sc_gather_t(tt1, tt2, e1_offsets.T, e2_offsets.T, d)
    return jnp.transpose(out_t, (2, 0, 1))
